# baseline (speedup 1.0000x reference)
import jax
import jax.numpy as jnp
from jax.experimental import pallas as pl
from jax.experimental.pallas import tpu as pltpu

INPUT_DIM = 128
TWO_D = 2 * INPUT_DIM
K = 512
B = 512
CHOICE_ALPHA = 0.001
GAMMA = 0.01
NUM_CLASSES = 10
C_PAD = 16

BT = 128
KC = 128
NC = K // KC
G = 2


def _body(x_ref, t_ref, counts_ref, comm_ref, labels_ref,
          out_ref, xn_scr, tT_scr, misc_scr, choice_scr):
    bb = pl.program_id(0)

    @pl.when(bb == 0)
    def _():
        xf = x_ref[...]
        mn = jnp.min(xf)
        mx = jnp.max(xf)
        xn_scr[...] = (xf - mn) / (mx - mn + 1e-10)

        ii = jax.lax.broadcasted_iota(jnp.int32, (K, K), 0)
        jj = jax.lax.broadcasted_iota(jnp.int32, (K, K), 1)
        eye = (ii == jj).astype(jnp.float32)
        tT_scr[...] = jax.lax.dot_general(
            t_ref[...], eye, (((0,), (0,)), ((), ())),
            preferred_element_type=jnp.float32,
            precision=jax.lax.Precision.HIGHEST,
        )
        s_t = jnp.sum(t_ref[...], axis=1, keepdims=True)
        cnt = counts_ref[...]
        cm = comm_ref[...]
        inv_col = 1.0 / (CHOICE_ALPHA + s_t + GAMMA * cnt)
        stage = jnp.concatenate([inv_col, cm], axis=1)
        misc_scr[0:2, :] = jax.lax.dot_general(
            stage, eye, (((0,), (0,)), ((), ())),
            preferred_element_type=jnp.float32,
            precision=jax.lax.Precision.HIGHEST,
        )

    @pl.when(bb > 0)
    def _choice_tile():
        _choice(bb - 1, xn_scr, tT_scr, misc_scr, choice_scr)

    @pl.when(bb == pl.num_programs(0) - 1)
    def _():
        cv = choice_scr[...]
        row_max = jnp.max(cv, axis=1, keepdims=True)
        iota_k = jax.lax.broadcasted_iota(jnp.int32, (B, K), 1)
        masked_idx = jnp.where(cv == row_max, iota_k, K)
        best = jnp.min(masked_idx, axis=1, keepdims=True)
        best_oh = (iota_k == best).astype(jnp.float32)

        cls_iota = jax.lax.broadcasted_iota(jnp.int32, (K, C_PAD), 1)
        lab_oh = (labels_ref[...] == cls_iota).astype(jnp.float32)

        cvz = jnp.where(misc_scr[1:2, :] > 0.0, cv, 0.0)
        cls_sums = jnp.dot(cvz, lab_oh, preferred_element_type=jnp.float32,
                           precision=jax.lax.Precision.HIGHEST)
        pred_oh = jnp.dot(best_oh, lab_oh, preferred_element_type=jnp.float32,
                          precision=jax.lax.Precision.HIGHEST)
        out_ref[...] = pred_oh * cls_sums


def _choice(tile, xn_scr, tT_scr, misc_scr, choice_scr):
    inv_denom = misc_scr[0:1, :]
    comm = misc_scr[1:2, :] > 0.0

    R = BT // 8
    for g in range(R // G):
        accg = [[jnp.zeros((8, KC), dtype=jnp.float32) for _ in range(NC)]
                for _ in range(G)]
        row0 = tile * BT + 8 * G * g
        for d in range(INPUT_DIM):
            a1s = []
            a2s = []
            for j in range(G):
                a1 = jax.lax.broadcast_in_dim(
                    xn_scr[pl.ds(row0 + 8 * j, 8), d : d + 1], (8, KC), (0, 1)
                )
                a1s.append(a1)
                a2s.append(1.0 - a1)
            for c in range(NC):
                b1 = jnp.broadcast_to(tT_scr[d : d + 1, c * KC : (c + 1) * KC], (8, KC))
                b2 = jnp.broadcast_to(
                    tT_scr[d + INPUT_DIM : d + INPUT_DIM + 1, c * KC : (c + 1) * KC],
                    (8, KC),
                )
                for j in range(G):
                    accg[j][c] = (
                        accg[j][c] + jnp.minimum(a1s[j], b1) + jnp.minimum(a2s[j], b2)
                    )
        for j in range(G):
            for c in range(NC):
                cvj = jnp.where(
                    comm[:, c * KC : (c + 1) * KC],
                    accg[j][c] * inv_denom[:, c * KC : (c + 1) * KC],
                    -jnp.inf,
                )
                choice_scr[pl.ds(row0 + 8 * j, 8), c * KC : (c + 1) * KC] = cvj


@jax.jit
def _run(x, templates, comm2d, labels2d, counts2d):
    logits_p = pl.pallas_call(
        _body,
        grid=(B // BT + 1,),
        in_specs=[
            pl.BlockSpec((B, INPUT_DIM), lambda bb: (0, 0)),
            pl.BlockSpec((K, TWO_D), lambda bb: (0, 0)),
            pl.BlockSpec((K, 1), lambda bb: (0, 0)),
            pl.BlockSpec((K, 1), lambda bb: (0, 0)),
            pl.BlockSpec((K, 1), lambda bb: (0, 0)),
        ],
        out_specs=pl.BlockSpec((B, C_PAD), lambda bb: (0, 0)),
        out_shape=jax.ShapeDtypeStruct((B, C_PAD), jnp.float32),
        scratch_shapes=[
            pltpu.VMEM((B, INPUT_DIM), jnp.float32),
            pltpu.VMEM((TWO_D, K), jnp.float32),
            pltpu.VMEM((8, K), jnp.float32),
            pltpu.VMEM((B, K), jnp.float32),
        ],
    )(x, templates, counts2d, comm2d, labels2d)
    return logits_p[:, :NUM_CLASSES]


def kernel(x, templates, committed, category_labels, category_counts, num_committed):
    comm2d = committed.astype(jnp.float32).reshape(K, 1)
    counts2d = category_counts.astype(jnp.float32).reshape(K, 1)
    labels2d = category_labels.reshape(K, 1)
    return _run(x, templates, comm2d, labels2d, counts2d)


# speedup vs baseline: 1.0317x; 1.0317x over previous
import jax
import jax.numpy as jnp
from jax.experimental import pallas as pl
from jax.experimental.pallas import tpu as pltpu

INPUT_DIM = 128
TWO_D = 2 * INPUT_DIM
K = 512
B = 512
CHOICE_ALPHA = 0.001
GAMMA = 0.01
NUM_CLASSES = 10
C_PAD = 16

BT = 256
KC = 128
NC = K // KC
G = 2


def _body(x_ref, t_ref, counts_ref, comm_ref, labels_ref,
          out_ref, xn_scr, tT_scr, misc_scr, choice_scr):
    bb = pl.program_id(0)

    @pl.when(bb == 0)
    def _():
        xf = x_ref[...]
        mn = jnp.min(xf)
        mx = jnp.max(xf)
        xn_scr[...] = (xf - mn) / (mx - mn + 1e-10)

        ii = jax.lax.broadcasted_iota(jnp.int32, (K, K), 0)
        jj = jax.lax.broadcasted_iota(jnp.int32, (K, K), 1)
        eye = (ii == jj).astype(jnp.float32)
        tT_scr[...] = jax.lax.dot_general(
            t_ref[...], eye, (((0,), (0,)), ((), ())),
            preferred_element_type=jnp.float32,
            precision=jax.lax.Precision.HIGHEST,
        )
        s_t = jnp.sum(t_ref[...], axis=1, keepdims=True)
        cnt = counts_ref[...]
        cm = comm_ref[...]
        inv_col = 1.0 / (CHOICE_ALPHA + s_t + GAMMA * cnt)
        stage = jnp.concatenate([inv_col, cm], axis=1)
        misc_scr[0:2, :] = jax.lax.dot_general(
            stage, eye, (((0,), (0,)), ((), ())),
            preferred_element_type=jnp.float32,
            precision=jax.lax.Precision.HIGHEST,
        )

    @pl.when(bb > 0)
    def _choice_tile():
        _choice(bb - 1, xn_scr, tT_scr, misc_scr, choice_scr)

    @pl.when(bb == pl.num_programs(0) - 1)
    def _():
        cv = choice_scr[...]
        row_max = jnp.max(cv, axis=1, keepdims=True)
        iota_k = jax.lax.broadcasted_iota(jnp.int32, (B, K), 1)
        masked_idx = jnp.where(cv == row_max, iota_k, K)
        best = jnp.min(masked_idx, axis=1, keepdims=True)
        best_oh = (iota_k == best).astype(jnp.float32)

        cls_iota = jax.lax.broadcasted_iota(jnp.int32, (K, C_PAD), 1)
        lab_oh = (labels_ref[...] == cls_iota).astype(jnp.float32)

        cvz = jnp.where(misc_scr[1:2, :] > 0.0, cv, 0.0)
        cls_sums = jnp.dot(cvz, lab_oh, preferred_element_type=jnp.float32,
                           precision=jax.lax.Precision.HIGHEST)
        pred_oh = jnp.dot(best_oh, lab_oh, preferred_element_type=jnp.float32,
                          precision=jax.lax.Precision.HIGHEST)
        out_ref[...] = pred_oh * cls_sums


def _choice(tile, xn_scr, tT_scr, misc_scr, choice_scr):
    inv_denom = misc_scr[0:1, :]
    comm = misc_scr[1:2, :] > 0.0

    R = BT // 8
    for g in range(R // G):
        accg = [[jnp.zeros((8, KC), dtype=jnp.float32) for _ in range(NC)]
                for _ in range(G)]
        row0 = tile * BT + 8 * G * g
        for d in range(INPUT_DIM):
            a1s = []
            a2s = []
            for j in range(G):
                a1 = jax.lax.broadcast_in_dim(
                    xn_scr[pl.ds(row0 + 8 * j, 8), d : d + 1], (8, KC), (0, 1)
                )
                a1s.append(a1)
                a2s.append(1.0 - a1)
            for c in range(NC):
                b1 = jnp.broadcast_to(tT_scr[d : d + 1, c * KC : (c + 1) * KC], (8, KC))
                b2 = jnp.broadcast_to(
                    tT_scr[d + INPUT_DIM : d + INPUT_DIM + 1, c * KC : (c + 1) * KC],
                    (8, KC),
                )
                for j in range(G):
                    accg[j][c] = (
                        accg[j][c] + jnp.minimum(a1s[j], b1) + jnp.minimum(a2s[j], b2)
                    )
        for j in range(G):
            for c in range(NC):
                cvj = jnp.where(
                    comm[:, c * KC : (c + 1) * KC],
                    accg[j][c] * inv_denom[:, c * KC : (c + 1) * KC],
                    -jnp.inf,
                )
                choice_scr[pl.ds(row0 + 8 * j, 8), c * KC : (c + 1) * KC] = cvj


@jax.jit
def _run(x, templates, comm2d, labels2d, counts2d):
    logits_p = pl.pallas_call(
        _body,
        grid=(B // BT + 1,),
        in_specs=[
            pl.BlockSpec((B, INPUT_DIM), lambda bb: (0, 0)),
            pl.BlockSpec((K, TWO_D), lambda bb: (0, 0)),
            pl.BlockSpec((K, 1), lambda bb: (0, 0)),
            pl.BlockSpec((K, 1), lambda bb: (0, 0)),
            pl.BlockSpec((K, 1), lambda bb: (0, 0)),
        ],
        out_specs=pl.BlockSpec((B, C_PAD), lambda bb: (0, 0)),
        out_shape=jax.ShapeDtypeStruct((B, C_PAD), jnp.float32),
        scratch_shapes=[
            pltpu.VMEM((B, INPUT_DIM), jnp.float32),
            pltpu.VMEM((TWO_D, K), jnp.float32),
            pltpu.VMEM((8, K), jnp.float32),
            pltpu.VMEM((B, K), jnp.float32),
        ],
    )(x, templates, counts2d, comm2d, labels2d)
    return logits_p[:, :NUM_CLASSES]


def kernel(x, templates, committed, category_labels, category_counts, num_committed):
    comm2d = committed.astype(jnp.float32).reshape(K, 1)
    counts2d = category_counts.astype(jnp.float32).reshape(K, 1)
    labels2d = category_labels.reshape(K, 1)
    return _run(x, templates, comm2d, labels2d, counts2d)


# speedup vs baseline: 1.0667x; 1.0339x over previous
import jax
import jax.numpy as jnp
from jax.experimental import pallas as pl
from jax.experimental.pallas import tpu as pltpu

INPUT_DIM = 128
TWO_D = 2 * INPUT_DIM
K = 512
B = 512
CHOICE_ALPHA = 0.001
GAMMA = 0.01
NUM_CLASSES = 10
C_PAD = 16

BT = 256
KC = 128
NC = K // KC
G = 4


def _body(x_ref, t_ref, counts_ref, comm_ref, labels_ref,
          out_ref, xn_scr, tT_scr, misc_scr, choice_scr):
    bb = pl.program_id(0)

    @pl.when(bb == 0)
    def _():
        xf = x_ref[...]
        mn = jnp.min(xf)
        mx = jnp.max(xf)
        xn_scr[...] = (xf - mn) / (mx - mn + 1e-10)

        ii = jax.lax.broadcasted_iota(jnp.int32, (K, K), 0)
        jj = jax.lax.broadcasted_iota(jnp.int32, (K, K), 1)
        eye = (ii == jj).astype(jnp.float32)
        tT_scr[...] = jax.lax.dot_general(
            t_ref[...], eye, (((0,), (0,)), ((), ())),
            preferred_element_type=jnp.float32,
            precision=jax.lax.Precision.HIGHEST,
        )
        s_t = jnp.sum(t_ref[...], axis=1, keepdims=True)
        cnt = counts_ref[...]
        cm = comm_ref[...]
        inv_col = 1.0 / (CHOICE_ALPHA + s_t + GAMMA * cnt)
        stage = jnp.concatenate([inv_col, cm], axis=1)
        misc_scr[0:2, :] = jax.lax.dot_general(
            stage, eye, (((0,), (0,)), ((), ())),
            preferred_element_type=jnp.float32,
            precision=jax.lax.Precision.HIGHEST,
        )

    @pl.when(bb > 0)
    def _choice_tile():
        _choice(bb - 1, xn_scr, tT_scr, misc_scr, choice_scr)

    @pl.when(bb == pl.num_programs(0) - 1)
    def _():
        cv = choice_scr[...]
        row_max = jnp.max(cv, axis=1, keepdims=True)
        iota_k = jax.lax.broadcasted_iota(jnp.int32, (B, K), 1)
        masked_idx = jnp.where(cv == row_max, iota_k, K)
        best = jnp.min(masked_idx, axis=1, keepdims=True)
        best_oh = (iota_k == best).astype(jnp.float32)

        cls_iota = jax.lax.broadcasted_iota(jnp.int32, (K, C_PAD), 1)
        lab_oh = (labels_ref[...] == cls_iota).astype(jnp.float32)

        cvz = jnp.where(misc_scr[1:2, :] > 0.0, cv, 0.0)
        cls_sums = jnp.dot(cvz, lab_oh, preferred_element_type=jnp.float32,
                           precision=jax.lax.Precision.HIGHEST)
        pred_oh = jnp.dot(best_oh, lab_oh, preferred_element_type=jnp.float32,
                          precision=jax.lax.Precision.HIGHEST)
        out_ref[...] = pred_oh * cls_sums


def _choice(tile, xn_scr, tT_scr, misc_scr, choice_scr):
    inv_denom = misc_scr[0:1, :]
    comm = misc_scr[1:2, :] > 0.0

    R = BT // 8
    for g in range(R // G):
        accg = [[jnp.zeros((8, KC), dtype=jnp.float32) for _ in range(NC)]
                for _ in range(G)]
        row0 = tile * BT + 8 * G * g
        for d in range(INPUT_DIM):
            a1s = []
            a2s = []
            for j in range(G):
                a1 = jax.lax.broadcast_in_dim(
                    xn_scr[pl.ds(row0 + 8 * j, 8), d : d + 1], (8, KC), (0, 1)
                )
                a1s.append(a1)
                a2s.append(1.0 - a1)
            for c in range(NC):
                b1 = jnp.broadcast_to(tT_scr[d : d + 1, c * KC : (c + 1) * KC], (8, KC))
                b2 = jnp.broadcast_to(
                    tT_scr[d + INPUT_DIM : d + INPUT_DIM + 1, c * KC : (c + 1) * KC],
                    (8, KC),
                )
                for j in range(G):
                    accg[j][c] = (
                        accg[j][c] + jnp.minimum(a1s[j], b1) + jnp.minimum(a2s[j], b2)
                    )
        for j in range(G):
            for c in range(NC):
                cvj = jnp.where(
                    comm[:, c * KC : (c + 1) * KC],
                    accg[j][c] * inv_denom[:, c * KC : (c + 1) * KC],
                    -jnp.inf,
                )
                choice_scr[pl.ds(row0 + 8 * j, 8), c * KC : (c + 1) * KC] = cvj


@jax.jit
def _run(x, templates, comm2d, labels2d, counts2d):
    logits_p = pl.pallas_call(
        _body,
        grid=(B // BT + 1,),
        in_specs=[
            pl.BlockSpec((B, INPUT_DIM), lambda bb: (0, 0)),
            pl.BlockSpec((K, TWO_D), lambda bb: (0, 0)),
            pl.BlockSpec((K, 1), lambda bb: (0, 0)),
            pl.BlockSpec((K, 1), lambda bb: (0, 0)),
            pl.BlockSpec((K, 1), lambda bb: (0, 0)),
        ],
        out_specs=pl.BlockSpec((B, C_PAD), lambda bb: (0, 0)),
        out_shape=jax.ShapeDtypeStruct((B, C_PAD), jnp.float32),
        scratch_shapes=[
            pltpu.VMEM((B, INPUT_DIM), jnp.float32),
            pltpu.VMEM((TWO_D, K), jnp.float32),
            pltpu.VMEM((8, K), jnp.float32),
            pltpu.VMEM((B, K), jnp.float32),
        ],
    )(x, templates, counts2d, comm2d, labels2d)
    return logits_p[:, :NUM_CLASSES]


def kernel(x, templates, committed, category_labels, category_counts, num_committed):
    comm2d = committed.astype(jnp.float32).reshape(K, 1)
    counts2d = category_counts.astype(jnp.float32).reshape(K, 1)
    labels2d = category_labels.reshape(K, 1)
    return _run(x, templates, comm2d, labels2d, counts2d)
